# baseline (speedup 1.0000x reference)
import functools

import jax
import jax.numpy as jnp
from jax import lax
from jax.experimental import pallas as pl
from jax.experimental.pallas import tpu as pltpu
from jax.experimental.pallas import tpu_sc as plsc

B, N, DIM = 16, 1024, 256
BN = B * N
K = 1024
M = 4096
NB = BN // M

NC, NS = 2, 16
NW = NC * NS
BPW = BN // NW
CH = 128
NCH = BPW // CH


def _tc_body(x_ref, e_ref, dist_ref, idx_ref, esq_ref):
    @pl.when(pl.program_id(0) == 0)
    def _():
        e0 = e_ref[...]
        esq_ref[...] = jnp.sum(e0 * e0, axis=1)[None, :]

    x = x_ref[...]
    e = e_ref[...]
    cross = jax.lax.dot_general(
        x, e, (((1,), (1,)), ((), ())), preferred_element_type=jnp.float32
    )
    x_sq = jnp.sum(x * x, axis=1, keepdims=True)
    dist = x_sq + esq_ref[...] - 2.0 * cross
    dist_ref[...] = dist
    m = jnp.min(dist, axis=1, keepdims=True)
    kiota = jax.lax.broadcasted_iota(jnp.int32, (M, K), 1)
    masked = jnp.where(dist == m, kiota, K)
    idx = jnp.min(masked, axis=1).astype(jnp.int32)
    idx_ref[...] = idx.reshape(1, 32, 128)


@functools.partial(
    pl.kernel,
    mesh=plsc.VectorSubcoreMesh(core_axis_name="c", subcore_axis_name="s"),
    out_type=jax.ShapeDtypeStruct((BN, DIM), jnp.float32),
    scratch_types=[
        pltpu.VMEM((BPW,), jnp.int32),
        pltpu.VMEM((CH, DIM), jnp.float32),
        pltpu.VMEM((CH, DIM), jnp.float32),
        pltpu.VMEM((CH, DIM), jnp.float32),
        pltpu.SemaphoreType.DMA,
        pltpu.SemaphoreType.DMA,
        pltpu.SemaphoreType.DMA,
        pltpu.SemaphoreType.DMA,
        pltpu.SemaphoreType.DMA,
        pltpu.SemaphoreType.DMA,
    ],
)
def _sc_gather(idx_hbm, e_hbm, out_hbm, idx_v, r0, r1, r2, g0, g1, g2, w0, w1, w2):
    wid = lax.axis_index("s") * NC + lax.axis_index("c")
    base = wid * BPW
    pltpu.sync_copy(idx_hbm.at[pl.ds(base, BPW)], idx_v)
    rows = (r0, r1, r2)
    gsem = (g0, g1, g2)
    wsem = (w0, w1, w2)
    gathers = [None] * NCH
    writes = [None] * NCH

    def start_gather(c):
        gathers[c] = pltpu.async_copy(
            e_hbm.at[idx_v.at[pl.ds(c * CH, CH)]], rows[c % 3], gsem[c % 3]
        )

    start_gather(0)
    if NCH > 1:
        start_gather(1)
    waited = set()
    for c in range(NCH):
        if c + 2 < NCH:
            if c - 1 >= 0:
                writes[c - 1].wait()
                waited.add(c - 1)
            start_gather(c + 2)
        gathers[c].wait()
        writes[c] = pltpu.async_copy(
            rows[c % 3], out_hbm.at[pl.ds(base + c * CH, CH)], wsem[c % 3]
        )
    for c in range(NCH):
        if c not in waited:
            writes[c].wait()


def kernel(x, embed):
    xf = x.reshape(BN, DIM)
    e = embed.reshape(K, DIM)
    dist, idx3 = pl.pallas_call(
        _tc_body,
        grid=(NB,),
        in_specs=[
            pl.BlockSpec((M, DIM), lambda i: (i, 0)),
            pl.BlockSpec((K, DIM), lambda i: (0, 0)),
        ],
        out_specs=[
            pl.BlockSpec((M, K), lambda i: (i, 0)),
            pl.BlockSpec((1, 32, 128), lambda i: (i, 0, 0)),
        ],
        out_shape=[
            jax.ShapeDtypeStruct((BN, K), jnp.float32),
            jax.ShapeDtypeStruct((NB, 32, 128), jnp.int32),
        ],
        scratch_shapes=[pltpu.VMEM((1, K), jnp.float32)],
    )(xf, e)
    idx = idx3.reshape(BN)
    q = _sc_gather(idx, e)
    return q.reshape(BN, 1, DIM), idx, dist


# speedup vs baseline: 1.0027x; 1.0027x over previous
import functools

import jax
import jax.numpy as jnp
from jax import lax
from jax.experimental import pallas as pl
from jax.experimental.pallas import tpu as pltpu
from jax.experimental.pallas import tpu_sc as plsc

B, N, DIM = 16, 1024, 256
BN = B * N
K = 1024
M = 2048
NB = BN // M

NC, NS = 2, 16
NW = NC * NS
BPW = BN // NW
CH = 128
NCH = BPW // CH


def _tc_body(x_ref, e_ref, dist_ref, idx_ref, esq_ref):
    @pl.when(pl.program_id(0) == 0)
    def _():
        e0 = e_ref[...]
        esq_ref[...] = jnp.sum(e0 * e0, axis=1)[None, :]

    x = x_ref[...]
    e = e_ref[...]
    cross = jax.lax.dot_general(
        x, e, (((1,), (1,)), ((), ())), preferred_element_type=jnp.float32
    )
    x_sq = jnp.sum(x * x, axis=1, keepdims=True)
    dist = x_sq + esq_ref[...] - 2.0 * cross
    dist_ref[...] = dist
    m = jnp.min(dist, axis=1, keepdims=True)
    kiota = jax.lax.broadcasted_iota(jnp.int32, (M, K), 1)
    masked = jnp.where(dist == m, kiota, K)
    idx = jnp.min(masked, axis=1).astype(jnp.int32)
    idx_ref[...] = idx.reshape(1, 16, 128)


@functools.partial(
    pl.kernel,
    mesh=plsc.VectorSubcoreMesh(core_axis_name="c", subcore_axis_name="s"),
    out_type=jax.ShapeDtypeStruct((BN, DIM), jnp.float32),
    scratch_types=[
        pltpu.VMEM((BPW,), jnp.int32),
        pltpu.VMEM((CH, DIM), jnp.float32),
        pltpu.VMEM((CH, DIM), jnp.float32),
        pltpu.VMEM((CH, DIM), jnp.float32),
        pltpu.SemaphoreType.DMA,
        pltpu.SemaphoreType.DMA,
        pltpu.SemaphoreType.DMA,
        pltpu.SemaphoreType.DMA,
        pltpu.SemaphoreType.DMA,
        pltpu.SemaphoreType.DMA,
    ],
)
def _sc_gather(idx_hbm, e_hbm, out_hbm, idx_v, r0, r1, r2, g0, g1, g2, w0, w1, w2):
    wid = lax.axis_index("s") * NC + lax.axis_index("c")
    base = wid * BPW
    pltpu.sync_copy(idx_hbm.at[pl.ds(base, BPW)], idx_v)
    rows = (r0, r1, r2)
    gsem = (g0, g1, g2)
    wsem = (w0, w1, w2)
    gathers = [None] * NCH
    writes = [None] * NCH

    def start_gather(c):
        gathers[c] = pltpu.async_copy(
            e_hbm.at[idx_v.at[pl.ds(c * CH, CH)]], rows[c % 3], gsem[c % 3]
        )

    start_gather(0)
    if NCH > 1:
        start_gather(1)
    waited = set()
    for c in range(NCH):
        if c + 2 < NCH:
            if c - 1 >= 0:
                writes[c - 1].wait()
                waited.add(c - 1)
            start_gather(c + 2)
        gathers[c].wait()
        writes[c] = pltpu.async_copy(
            rows[c % 3], out_hbm.at[pl.ds(base + c * CH, CH)], wsem[c % 3]
        )
    for c in range(NCH):
        if c not in waited:
            writes[c].wait()


def kernel(x, embed):
    xf = x.reshape(BN, DIM)
    e = embed.reshape(K, DIM)
    dist, idx3 = pl.pallas_call(
        _tc_body,
        grid=(NB,),
        in_specs=[
            pl.BlockSpec((M, DIM), lambda i: (i, 0)),
            pl.BlockSpec((K, DIM), lambda i: (0, 0)),
        ],
        out_specs=[
            pl.BlockSpec((M, K), lambda i: (i, 0)),
            pl.BlockSpec((1, 16, 128), lambda i: (i, 0, 0)),
        ],
        out_shape=[
            jax.ShapeDtypeStruct((BN, K), jnp.float32),
            jax.ShapeDtypeStruct((NB, 16, 128), jnp.int32),
        ],
        scratch_shapes=[pltpu.VMEM((1, K), jnp.float32)],
    )(xf, e)
    idx = idx3.reshape(BN)
    q = _sc_gather(idx, e)
    return q.reshape(BN, 1, DIM), idx, dist


# speedup vs baseline: 1.0038x; 1.0010x over previous
import functools

import jax
import jax.numpy as jnp
from jax import lax
from jax.experimental import pallas as pl
from jax.experimental.pallas import tpu as pltpu
from jax.experimental.pallas import tpu_sc as plsc

B, N, DIM = 16, 1024, 256
BN = B * N
K = 1024
M = 4096
NB = BN // M

NC, NS = 2, 16
NW = NC * NS
BPW = BN // NW
CH = 128
NCH = BPW // CH


def _tc_body(x_ref, e_ref, dist_ref, idx_ref, esq_ref):
    @pl.when(pl.program_id(0) == 0)
    def _():
        e0 = e_ref[...]
        esq_ref[...] = jnp.sum(e0 * e0, axis=1)[None, :]

    x = x_ref[...]
    e = e_ref[...]
    cross = jax.lax.dot_general(
        x, e, (((1,), (1,)), ((), ())), preferred_element_type=jnp.float32
    )
    x_sq = jnp.sum(x * x, axis=1, keepdims=True)
    dist = x_sq + esq_ref[...] - 2.0 * cross
    dist_ref[...] = dist
    m = jnp.min(dist, axis=1, keepdims=True)
    kiota = jax.lax.broadcasted_iota(jnp.int32, (M, K), 1)
    masked = jnp.where(dist == m, kiota, K)
    idx = jnp.min(masked, axis=1).astype(jnp.int32)
    idx_ref[...] = idx.reshape(1, 32, 128)


@functools.partial(
    pl.kernel,
    mesh=plsc.VectorSubcoreMesh(core_axis_name="c", subcore_axis_name="s"),
    out_type=jax.ShapeDtypeStruct((BN, DIM), jnp.float32),
    scratch_types=[
        pltpu.VMEM((BPW,), jnp.int32),
        pltpu.VMEM((CH, DIM), jnp.float32),
        pltpu.VMEM((CH, DIM), jnp.float32),
        pltpu.VMEM((CH, DIM), jnp.float32),
        pltpu.SemaphoreType.DMA,
        pltpu.SemaphoreType.DMA,
        pltpu.SemaphoreType.DMA,
        pltpu.SemaphoreType.DMA,
        pltpu.SemaphoreType.DMA,
        pltpu.SemaphoreType.DMA,
    ],
)
def _sc_gather(idx_hbm, e_hbm, out_hbm, idx_v, r0, r1, r2, g0, g1, g2, w0, w1, w2):
    wid = lax.axis_index("s") * NC + lax.axis_index("c")
    base = wid * BPW
    pltpu.sync_copy(idx_hbm.at[pl.ds(base, BPW)], idx_v)
    rows = (r0, r1, r2)
    gsem = (g0, g1, g2)
    wsem = (w0, w1, w2)
    gathers = [None] * NCH
    writes = [None] * NCH

    def start_gather(c):
        gathers[c] = pltpu.async_copy(
            e_hbm.at[idx_v.at[pl.ds(c * CH, CH)]], rows[c % 3], gsem[c % 3]
        )

    start_gather(0)
    if NCH > 1:
        start_gather(1)
    waited = set()
    for c in range(NCH):
        if c + 2 < NCH:
            if c - 1 >= 0:
                writes[c - 1].wait()
                waited.add(c - 1)
            start_gather(c + 2)
        gathers[c].wait()
        writes[c] = pltpu.async_copy(
            rows[c % 3], out_hbm.at[pl.ds(base + c * CH, CH)], wsem[c % 3]
        )
    for c in range(NCH):
        if c not in waited:
            writes[c].wait()


def kernel(x, embed):
    xf = x.reshape(BN, DIM)
    e = embed.reshape(K, DIM)
    dist, idx3 = pl.pallas_call(
        _tc_body,
        grid=(NB,),
        in_specs=[
            pl.BlockSpec((M, DIM), lambda i: (i, 0)),
            pl.BlockSpec((K, DIM), lambda i: (0, 0)),
        ],
        out_specs=[
            pl.BlockSpec((M, K), lambda i: (i, 0)),
            pl.BlockSpec((1, 32, 128), lambda i: (i, 0, 0)),
        ],
        out_shape=[
            jax.ShapeDtypeStruct((BN, K), jnp.float32),
            jax.ShapeDtypeStruct((NB, 32, 128), jnp.int32),
        ],
        scratch_shapes=[pltpu.VMEM((1, K), jnp.float32)],
    )(xf, e)
    idx = idx3.reshape(BN)
    q = _sc_gather(idx, e)
    return q.reshape(BN, 1, DIM), idx, dist
